# baseline (speedup 1.0000x reference)
import functools

import jax
import jax.numpy as jnp
from jax import lax
from jax.experimental import pallas as pl
from jax.experimental.pallas import tpu as pltpu
from jax.experimental.pallas import tpu_sc as plsc

NUM_EMB = 1_000_000
D = 64
SEQ = 200
BATCH = 4096
HALF = 100
NITEMS = BATCH * SEQ // HALF


def _make_kernel(num_workers):
    items_per_w = NITEMS // num_workers
    mesh = plsc.VectorSubcoreMesh(core_axis_name="c", subcore_axis_name="s")

    @functools.partial(
        pl.kernel,
        out_type=jax.ShapeDtypeStruct((NITEMS, HALF, D), jnp.float32),
        mesh=mesh,
        scratch_types=[
            pltpu.VMEM((HALF, D), jnp.float32),
            pltpu.VMEM((HALF,), jnp.int32),
            pltpu.VMEM((HALF, D), jnp.float32),
            pltpu.SemaphoreType.DMA,
        ],
        compiler_params=pltpu.CompilerParams(use_tc_tiling_on_sc=False),
    )
    def body(idx_hbm, table_hbm, pos_hbm, out_hbm, posw_v, idx_v, tok_v, sem):
        nc = 2
        wid = lax.axis_index("s") * nc + lax.axis_index("c")
        parity = lax.rem(wid, 2)
        pltpu.sync_copy(pos_hbm.at[parity], posw_v)

        def per_item(k, carry):
            t = wid + k * num_workers
            pltpu.sync_copy(idx_hbm.at[t], idx_v)
            pltpu.async_copy(table_hbm.at[idx_v], tok_v, sem).wait()

            def add_row(r, c):
                for j in range(D // 16):
                    sl = pl.ds(j * 16, 16)
                    tok_v[r, sl] = tok_v[r, sl] + posw_v[r, sl]
                return c

            lax.fori_loop(0, HALF, add_row, 0)
            pltpu.sync_copy(tok_v, out_hbm.at[t])
            return carry

        lax.fori_loop(0, items_per_w, per_item, 0)

    return body


def kernel(inputs, token_table, pos_table):
    idx = inputs.reshape(NITEMS, HALF).astype(jnp.int32)
    pos3 = pos_table.reshape(SEQ // HALF, HALF, D)
    info = plsc.get_sparse_core_info()
    nw = info.num_cores * info.num_subcores
    out = _make_kernel(nw)(idx, token_table, pos3)
    return out.reshape(BATCH, SEQ, D)
